# Initial kernel scaffold; baseline (speedup 1.0000x reference)
import jax
import jax.numpy as jnp
from jax.experimental import pallas as pl


def kernel(x, attention_mask, input_ids):
    raise NotImplementedError("write your pallas kernel here")


# speedup vs baseline: 7.1821x; 7.1821x over previous
import jax
import jax.numpy as jnp
from jax import lax
from jax.experimental import pallas as pl

START_MIN, START_MAX = 1, 2
END_MIN, END_MAX = 3, 4
N_SEG = 16


def _incl_cumsum_lanes(a):
    s = a.shape[1]
    sh = 1
    while sh < s:
        shifted = jnp.concatenate(
            [jnp.zeros((1, sh), a.dtype), a[:, : s - sh]], axis=1)
        a = a + shifted
        sh *= 2
    return a


def _row_kernel(ids_ref, x_ref, o_ref):
    ids = ids_ref[0]
    s = ids.shape[1]
    sm = (ids >= START_MIN) & (ids <= START_MAX)
    em = (ids >= END_MIN) & (ids <= END_MAX)
    s_cum = _incl_cumsum_lanes(sm.astype(jnp.int32))
    e_cum = _incl_cumsum_lanes(em.astype(jnp.int32))

    r = lax.broadcasted_iota(jnp.int32, (3 * N_SEG, s), 0)
    seg = r // 3
    c = r % 3
    s_eq = s_cum == seg + 1
    is_start = jnp.where(sm & s_eq, 1.0, 0.0)
    is_end = jnp.where(em & (e_cum == seg + 1), 1.0, 0.0)
    is_interior = jnp.where(s_eq & (e_cum == seg) & (~sm) & (~em), 1.0, 0.0)
    maskf = jnp.where(c == 0, is_start,
                      jnp.where(c == 1, is_end, is_interior))

    sums = lax.dot_general(
        maskf, x_ref[0],
        dimension_numbers=(((1,), (0,)), ((), ())),
        preferred_element_type=jnp.float32,
    )
    counts = jnp.sum(maskf, axis=1, keepdims=True)
    cl = lax.broadcasted_iota(jnp.int32, (3 * N_SEG, 1), 0)
    div = jnp.where(cl % 3 == 2, counts, 1.0)
    o_ref[0] = sums / div


def kernel(x, attention_mask, input_ids):
    del attention_mask
    b, s, d = x.shape
    ids3 = input_ids.reshape(b, 1, s)
    out = pl.pallas_call(
        _row_kernel,
        grid=(b,),
        in_specs=[
            pl.BlockSpec((1, 1, s), lambda i: (i, 0, 0)),
            pl.BlockSpec((1, s, d), lambda i: (i, 0, 0)),
        ],
        out_specs=pl.BlockSpec((1, 3 * N_SEG, d), lambda i: (i, 0, 0)),
        out_shape=jax.ShapeDtypeStruct((b, 3 * N_SEG, d), jnp.float32),
    )(ids3, x)
    return out.reshape(b, N_SEG, 3 * d)
